# initial kernel scaffold (unmeasured)
import jax
import jax.numpy as jnp
from jax import lax
from jax.experimental import pallas as pl
from jax.experimental.pallas import tpu as pltpu

N_DEV = 8
_GELU_C = 0.7978845608028654


def _gelu(y):
    return 0.5 * y * (1.0 + jnp.tanh(_GELU_C * (y + 0.044715 * y * y * y)))


def kernel(x, w_mat):
    m, _ = x.shape
    _, n = w_mat.shape
    chunk = m // N_DEV

    def body(x_ref, w_ref, out_ref, comm_ref, send_sems, recv_sems, credit_sem):
        my = lax.axis_index("i")
        left = lax.rem(my + N_DEV - 1, N_DEV)
        right = lax.rem(my + 1, N_DEV)

        barrier = pltpu.get_barrier_semaphore()
        for nbr in (left, right):
            pl.semaphore_signal(barrier, inc=1, device_id=(nbr,),
                                device_id_type=pl.DeviceIdType.MESH)
        pl.semaphore_wait(barrier, 2)

        def partial_chunk(c):
            rows = x_ref[pl.ds(c * chunk, chunk), :]
            return jnp.dot(rows, w_ref[...], preferred_element_type=jnp.float32)

        comm_ref[0, :, :] = partial_chunk(my)

        n_hops = 2 * (N_DEV - 1)
        for h in range(n_hops):
            s, r = h % 2, (h + 1) % 2
            rdma = pltpu.make_async_remote_copy(
                src_ref=comm_ref.at[s],
                dst_ref=comm_ref.at[r],
                send_sem=send_sems.at[s],
                recv_sem=recv_sems.at[r],
                device_id=(right,),
                device_id_type=pl.DeviceIdType.MESH,
            )
            if h >= 1:
                pl.semaphore_wait(credit_sem, 1)
            rdma.start()
            rdma.wait()
            if h < n_hops - 1:
                pl.semaphore_signal(credit_sem, inc=1, device_id=(left,),
                                    device_id_type=pl.DeviceIdType.MESH)
            if h < N_DEV - 1:
                c = lax.rem(my + 2 * N_DEV - h - 1, N_DEV)
                acc = comm_ref[r, :, :] + partial_chunk(c)
                if h == N_DEV - 2:
                    acc = _gelu(acc)
                    out_ref[pl.ds(right * chunk, chunk), :] = acc
                comm_ref[r, :, :] = acc
            else:
                g = h - (N_DEV - 1)
                origin = lax.rem(my + 2 * N_DEV - g, N_DEV)
                out_ref[pl.ds(origin * chunk, chunk), :] = comm_ref[r, :, :]

    return pl.pallas_call(
        body,
        out_shape=jax.ShapeDtypeStruct((m, n), jnp.float32),
        in_specs=[
            pl.BlockSpec(memory_space=pltpu.VMEM),
            pl.BlockSpec(memory_space=pltpu.VMEM),
        ],
        out_specs=pl.BlockSpec(memory_space=pltpu.VMEM),
        scratch_shapes=[
            pltpu.VMEM((2, chunk, n), jnp.float32),
            pltpu.SemaphoreType.DMA((2,)),
            pltpu.SemaphoreType.DMA((2,)),
            pltpu.SemaphoreType.REGULAR,
        ],
        compiler_params=pltpu.CompilerParams(collective_id=0),
    )(x, w_mat)


# baseline (device time: 715336 ns/iter reference)
import jax
import jax.numpy as jnp
from jax import lax
from jax.experimental import pallas as pl
from jax.experimental.pallas import tpu as pltpu

N_DEV = 8
_GELU_C = 0.7978845608028654


def _gelu(y):
    return 0.5 * y * (1.0 + jnp.tanh(_GELU_C * (y + 0.044715 * y * y * y)))


def kernel(x, w_mat):
    m, _ = x.shape
    _, n = w_mat.shape
    chunk = m // N_DEV

    def body(x_ref, w_ref, out_ref, comm_ref, send_sems, recv_sems, credit_sem,
             store_sem):
        my = lax.axis_index("i")
        left = lax.rem(my + N_DEV - 1, N_DEV)
        right = lax.rem(my + 1, N_DEV)

        barrier = pltpu.get_barrier_semaphore()
        for nbr in (left, right):
            pl.semaphore_signal(barrier, inc=1, device_id=(nbr,),
                                device_id_type=pl.DeviceIdType.MESH)
        pl.semaphore_wait(barrier, 2)

        def partial_chunk(c):
            rows = x_ref[pl.ds(c * chunk, chunk), :]
            return jnp.dot(rows, w_ref[...], preferred_element_type=jnp.float32)

        comm_ref[0, :, :] = partial_chunk(my)

        n_hops = 2 * (N_DEV - 1)
        for h in range(n_hops):
            s, r = h % 2, (h + 1) % 2
            rdma = pltpu.make_async_remote_copy(
                src_ref=comm_ref.at[s],
                dst_ref=comm_ref.at[r],
                send_sem=send_sems.at[s],
                recv_sem=recv_sems.at[r],
                device_id=(right,),
                device_id_type=pl.DeviceIdType.MESH,
            )
            if h >= 1:
                pl.semaphore_wait(credit_sem, 1)
            rdma.start()
            rdma.wait()
            if h < n_hops - 1:
                pl.semaphore_signal(credit_sem, inc=1, device_id=(left,),
                                    device_id_type=pl.DeviceIdType.MESH)
            if h < N_DEV - 1:
                c = lax.rem(my + 2 * N_DEV - h - 1, N_DEV)
                acc = comm_ref[r, :, :] + partial_chunk(c)
                if h == N_DEV - 2:
                    acc = _gelu(acc)
                comm_ref[r, :, :] = acc
                if h == N_DEV - 2:
                    store = pltpu.make_async_copy(
                        comm_ref.at[r],
                        out_ref.at[pl.ds(right * chunk, chunk), :],
                        store_sem,
                    )
                    store.start()
                    store.wait()
            else:
                g = h - (N_DEV - 1)
                origin = lax.rem(my + 2 * N_DEV - g, N_DEV)
                store = pltpu.make_async_copy(
                    comm_ref.at[r],
                    out_ref.at[pl.ds(origin * chunk, chunk), :],
                    store_sem,
                )
                store.start()
                store.wait()

    return pl.pallas_call(
        body,
        out_shape=jax.ShapeDtypeStruct((m, n), jnp.float32),
        in_specs=[
            pl.BlockSpec(memory_space=pltpu.VMEM),
            pl.BlockSpec(memory_space=pltpu.VMEM),
        ],
        out_specs=pl.BlockSpec(memory_space=pl.ANY),
        scratch_shapes=[
            pltpu.VMEM((2, chunk, n), jnp.float32),
            pltpu.SemaphoreType.DMA((2,)),
            pltpu.SemaphoreType.DMA((2,)),
            pltpu.SemaphoreType.REGULAR,
            pltpu.SemaphoreType.DMA,
        ],
        compiler_params=pltpu.CompilerParams(
            collective_id=0,
            vmem_limit_bytes=100 * 1024 * 1024,
        ),
    )(x, w_mat)


# device time: 383864 ns/iter; 1.8635x vs baseline; 1.8635x over previous
import jax
import jax.numpy as jnp
from jax import lax
from jax.experimental import pallas as pl
from jax.experimental.pallas import tpu as pltpu

N_DEV = 8
_GELU_C = 0.7978845608028654


def _gelu(y):
    return 0.5 * y * (1.0 + jnp.tanh(_GELU_C * (y + 0.044715 * y * y * y)))


def kernel(x, w_mat):
    m, _ = x.shape
    _, n = w_mat.shape
    chunk = m // N_DEV
    half = chunk // 2

    n_hops = 2 * (N_DEV - 1)

    def body(x_ref, w_ref, out_ref, comm_cw, comm_ccw,
             send_cw, recv_cw, send_ccw, recv_ccw,
             store_cw, store_ccw, credit_cw, credit_ccw):
        my = lax.axis_index("i")
        left = lax.rem(my + N_DEV - 1, N_DEV)
        right = lax.rem(my + 1, N_DEV)

        barrier = pltpu.get_barrier_semaphore()
        for nbr in (left, right):
            pl.semaphore_signal(barrier, inc=1, device_id=(nbr,),
                                device_id_type=pl.DeviceIdType.MESH)
        pl.semaphore_wait(barrier, 2)

        def ptop(c):
            rows = x_ref[pl.ds(c * chunk, half), :]
            return jnp.dot(rows, w_ref[...], preferred_element_type=jnp.float32)

        def pbot(c):
            rows = x_ref[pl.ds(c * chunk + half, half), :]
            return jnp.dot(rows, w_ref[...], preferred_element_type=jnp.float32)

        comm_cw[0, :, :] = ptop(my)
        comm_ccw[0, :, :] = pbot(my)

        pending_stores = []
        prev_stores = None
        for h in range(n_hops):
            s, r = h % 4, (h + 1) % 4
            if h >= 3:
                pl.semaphore_wait(credit_cw, 1)
                pl.semaphore_wait(credit_ccw, 1)
            rd_cw = pltpu.make_async_remote_copy(
                src_ref=comm_cw.at[s], dst_ref=comm_cw.at[r],
                send_sem=send_cw.at[s], recv_sem=recv_cw.at[r],
                device_id=(right,), device_id_type=pl.DeviceIdType.MESH,
            )
            rd_ccw = pltpu.make_async_remote_copy(
                src_ref=comm_ccw.at[s], dst_ref=comm_ccw.at[r],
                send_sem=send_ccw.at[s], recv_sem=recv_ccw.at[r],
                device_id=(left,), device_id_type=pl.DeviceIdType.MESH,
            )
            rd_cw.start()
            rd_ccw.start()

            c_cw = lax.rem(my + 2 * N_DEV - h - 1, N_DEV)
            c_ccw = lax.rem(my + h + 1, N_DEV)

            if h < N_DEV - 1:
                pt = ptop(c_cw)
                pb = pbot(c_ccw)

            rd_cw.wait_recv()
            rd_ccw.wait_recv()

            if h < N_DEV - 1:
                acc_t = comm_cw[r, :, :] + pt
                acc_b = comm_ccw[r, :, :] + pb
                if h == N_DEV - 2:
                    acc_t = _gelu(acc_t)
                    acc_b = _gelu(acc_b)
                comm_cw[r, :, :] = acc_t
                comm_ccw[r, :, :] = acc_b

            if h >= N_DEV - 2:
                st_cw = pltpu.make_async_copy(
                    comm_cw.at[r],
                    out_ref.at[pl.ds(c_cw * chunk, half), :],
                    store_cw.at[r],
                )
                st_ccw = pltpu.make_async_copy(
                    comm_ccw.at[r],
                    out_ref.at[pl.ds(c_ccw * chunk + half, half), :],
                    store_ccw.at[r],
                )
                st_cw.start()
                st_ccw.start()
            else:
                st_cw = st_ccw = None

            rd_cw.wait_send()
            rd_ccw.wait_send()

            if h <= 10:
                if prev_stores is not None:
                    prev_stores[0].wait()
                    prev_stores[1].wait()
                pl.semaphore_signal(credit_cw, inc=1, device_id=(left,),
                                    device_id_type=pl.DeviceIdType.MESH)
                pl.semaphore_signal(credit_ccw, inc=1, device_id=(right,),
                                    device_id_type=pl.DeviceIdType.MESH)
            elif st_cw is not None:
                pending_stores.append((st_cw, st_ccw))
                st_cw = st_ccw = None
            if st_cw is not None and h == 10:
                pending_stores.append((st_cw, st_ccw))
                st_cw = st_ccw = None
            prev_stores = (st_cw, st_ccw) if st_cw is not None else None

        for st_a, st_b in pending_stores:
            st_a.wait()
            st_b.wait()

    return pl.pallas_call(
        body,
        out_shape=jax.ShapeDtypeStruct((m, n), jnp.float32),
        in_specs=[
            pl.BlockSpec(memory_space=pltpu.VMEM),
            pl.BlockSpec(memory_space=pltpu.VMEM),
        ],
        out_specs=pl.BlockSpec(memory_space=pl.ANY),
        scratch_shapes=[
            pltpu.VMEM((4, half, n), jnp.float32),
            pltpu.VMEM((4, half, n), jnp.float32),
            pltpu.SemaphoreType.DMA((4,)),
            pltpu.SemaphoreType.DMA((4,)),
            pltpu.SemaphoreType.DMA((4,)),
            pltpu.SemaphoreType.DMA((4,)),
            pltpu.SemaphoreType.DMA((4,)),
            pltpu.SemaphoreType.DMA((4,)),
            pltpu.SemaphoreType.REGULAR,
            pltpu.SemaphoreType.REGULAR,
        ],
        compiler_params=pltpu.CompilerParams(
            collective_id=0,
            vmem_limit_bytes=100 * 1024 * 1024,
        ),
    )(x, w_mat)
